# baseline (speedup 1.0000x reference)
import functools

import numpy as np
import jax
import jax.numpy as jnp
from jax import lax
from jax.experimental import pallas as pl
from jax.experimental.pallas import tpu as pltpu
from jax.experimental.pallas import tpu_sc as plsc

L = 16
T = 65536
B = 16384
PRIMES_I32 = [int(np.uint32(p).astype(np.int32)) for p in
              (1, 2654435761, 805459861, 3674653429)]
RES_LIST = [float(np.floor(16.0 * 1.5 ** l)) for l in range(L)]

NC, NS = 2, 16
NW = NC * NS
TASKS_PER_TILE = 64 // NW
CS = 4096
NSTEP = CS // 16
PCHUNK = 16384
NPACK = 2 * T // PCHUNK


def _sc_encode_body(tabf_hbm, zt_hbm, out_hbm,
                    tab_v, stg_v, z_v, o0_v, o1_v):
    wid = lax.axis_index("s") * NC + lax.axis_index("c")
    iota2 = 2 * lax.iota(jnp.int32, 16)

    for j in range(TASKS_PER_TILE):
        task = wid * TASKS_PER_TILE + j
        grp = lax.shift_right_logical(task, 4)
        lvl = lax.bitwise_and(task, 15)
        res = jnp.float32(0.0)
        for k in range(L):
            res = jnp.where(lvl == k, jnp.float32(RES_LIST[k]), res)

        for pc in range(NPACK):
            pltpu.sync_copy(tabf_hbm.at[task, pl.ds(pc * PCHUNK, PCHUNK)],
                            stg_v)

            @plsc.parallel_loop(0, PCHUNK // 32, unroll=2)
            def pack_step(s):
                ev = iota2 + s * 32
                v0 = plsc.load_gather(stg_v, [ev])
                v1 = plsc.load_gather(stg_v, [ev + 1])
                w = plsc.bitcast(
                    plsc.pack(v0, v1, format=plsc.PackFormat.INTERLEAVED),
                    jnp.int32)
                tab_v[pl.ds(pc * (PCHUNK // 2) + s * 16, 16)] = w

        for c in range(B // CS):
            pltpu.sync_copy(
                zt_hbm.at[pl.ds(grp * 4, 4), pl.ds(c * CS, CS)], z_v)

            @plsc.parallel_loop(0, NSTEP, unroll=2)
            def step(s):
                off = pl.multiple_of(s * 16, 16)
                fr = []
                om = []
                a = []
                b = []
                for dd in range(4):
                    zd = z_v[dd, pl.ds(off, 16)]
                    x = 1.0 / (1.0 + jnp.exp(-zd))
                    pos = x * res
                    pi = pos.astype(jnp.int32)
                    fd = pos - pi.astype(jnp.float32)
                    fr.append(fd)
                    om.append(1.0 - fd)
                    if dd == 0:
                        ad = pi
                        bd = pi + 1
                    else:
                        ad = pi * jnp.int32(PRIMES_I32[dd])
                        bd = ad + jnp.int32(PRIMES_I32[dd])
                    a.append(lax.bitwise_and(ad, jnp.int32(0xFFFF)))
                    b.append(lax.bitwise_and(bd, jnp.int32(0xFFFF)))
                h01 = [(b[0] if (lo & 1) else a[0]) ^
                       (b[1] if (lo >> 1) else a[1]) for lo in range(4)]
                w01 = [(fr[0] if (lo & 1) else om[0]) *
                       (fr[1] if (lo >> 1) else om[1]) for lo in range(4)]
                h23 = [(b[2] if (hi & 1) else a[2]) ^
                       (b[3] if (hi >> 1) else a[3]) for hi in range(4)]
                w23 = [(fr[2] if (hi & 1) else om[2]) *
                       (fr[3] if (hi >> 1) else om[3]) for hi in range(4)]
                acc0 = jnp.zeros((16,), jnp.float32)
                acc1 = jnp.zeros((16,), jnp.float32)
                for cj in range(16):
                    idx = h01[cj & 3] ^ h23[cj >> 2]
                    w = w01[cj & 3] * w23[cj >> 2]
                    word = plsc.load_gather(tab_v, [idx])
                    f0 = plsc.bitcast(lax.shift_left(word, jnp.int32(16)),
                                      jnp.float32)
                    f1 = plsc.bitcast(lax.bitwise_and(word, jnp.int32(-65536)),
                                      jnp.float32)
                    acc0 = acc0 + w * f0
                    acc1 = acc1 + w * f1
                o0_v[pl.ds(off, 16)] = acc0
                o1_v[pl.ds(off, 16)] = acc1

            pltpu.sync_copy(o0_v, out_hbm.at[task * 2, pl.ds(c * CS, CS)])
            pltpu.sync_copy(o1_v, out_hbm.at[task * 2 + 1, pl.ds(c * CS, CS)])


_sc_encode = functools.partial(
    pl.kernel,
    out_type=jax.ShapeDtypeStruct((128, B), jnp.float32),
    mesh=plsc.VectorSubcoreMesh(core_axis_name="c", subcore_axis_name="s"),
    compiler_params=pltpu.CompilerParams(needs_layout_passes=False),
    scratch_types=[
        pltpu.VMEM((T,), jnp.int32),
        pltpu.VMEM((PCHUNK,), jnp.float32),
        pltpu.VMEM((4, CS), jnp.float32),
        pltpu.VMEM((CS,), jnp.float32),
        pltpu.VMEM((CS,), jnp.float32),
    ],
)(_sc_encode_body)


def _mlp_body(e_ref, w1_ref, b1_ref, w2_ref, b2_ref, o_ref):
    e = e_ref[...]
    h = lax.dot_general(w1_ref[...], e, (((0,), (0,)), ((), ())),
                        preferred_element_type=jnp.float32)
    h = h + b1_ref[...]
    h = jnp.where(h >= 0, h, 0.01 * h)
    lt = lax.dot_general(w2_ref[...], h, (((0,), (0,)), ((), ())),
                         preferred_element_type=jnp.float32)
    lt = lt + b2_ref[...]
    o_ref[...] = lt.T


def _mlp(enc, W1, b1c, W2, b2c):
    bsz = 2048
    return pl.pallas_call(
        _mlp_body,
        grid=(B // bsz,),
        in_specs=[
            pl.BlockSpec((128, bsz), lambda i: (0, i)),
            pl.BlockSpec((128, 256), lambda i: (0, 0)),
            pl.BlockSpec((256, 1), lambda i: (0, 0)),
            pl.BlockSpec((256, 64), lambda i: (0, 0)),
            pl.BlockSpec((64, 1), lambda i: (0, 0)),
        ],
        out_specs=pl.BlockSpec((bsz, 64), lambda i: (i, 0)),
        out_shape=jax.ShapeDtypeStruct((B, 64), jnp.float32),
    )(enc, W1, b1c, W2, b2c)


def kernel(z, tables, W1, b1, W2, b2):
    zt = z.T
    tabf = tables.reshape(64, 2 * T)
    enc = _sc_encode(tabf, zt)
    return _mlp(enc, W1, b1.reshape(256, 1), W2, b2.reshape(64, 1))


# speedup vs baseline: 2.1601x; 2.1601x over previous
import functools

import numpy as np
import jax
import jax.numpy as jnp
from jax import lax
from jax.experimental import pallas as pl
from jax.experimental.pallas import tpu as pltpu
from jax.experimental.pallas import tpu_sc as plsc

L = 16
T = 65536
B = 16384
PRIMES_I32 = [int(np.uint32(p).astype(np.int32)) for p in
              (1, 2654435761, 805459861, 3674653429)]
RES_LIST = [float(np.floor(16.0 * 1.5 ** l)) for l in range(L)]

NC, NS = 2, 16
NW = NC * NS
TASKS_PER_TILE = 64 // NW
CS = 8192
NSTEP = CS // 16


def _sc_encode_body(tabf_hbm, zt_hbm, out_hbm,
                    tab_v, z_v, o0_v, o1_v):
    wid = lax.axis_index("s") * NC + lax.axis_index("c")

    for j in range(TASKS_PER_TILE):
        task = wid * TASKS_PER_TILE + j
        grp = lax.shift_right_logical(task, 4)
        lvl = lax.bitwise_and(task, 15)
        res = jnp.float32(0.0)
        for k in range(L):
            res = jnp.where(lvl == k, jnp.float32(RES_LIST[k]), res)

        pltpu.sync_copy(tabf_hbm.at[task], tab_v)

        for c in range(B // CS):
            pltpu.sync_copy(
                zt_hbm.at[pl.ds(grp * 4, 4), pl.ds(c * CS, CS)], z_v)

            @plsc.parallel_loop(0, NSTEP, unroll=2)
            def step(s):
                off = pl.multiple_of(s * 16, 16)
                fr = []
                om = []
                a = []
                b = []
                for dd in range(4):
                    zd = z_v[dd, pl.ds(off, 16)]
                    x = 1.0 / (1.0 + jnp.exp(-zd))
                    pos = x * res
                    pi = pos.astype(jnp.int32)
                    fd = pos - pi.astype(jnp.float32)
                    fr.append(fd)
                    om.append(1.0 - fd)
                    if dd == 0:
                        ad = pi
                        bd = pi + 1
                    else:
                        ad = pi * jnp.int32(PRIMES_I32[dd])
                        bd = ad + jnp.int32(PRIMES_I32[dd])
                    a.append(lax.bitwise_and(ad, jnp.int32(0xFFFF)))
                    b.append(lax.bitwise_and(bd, jnp.int32(0xFFFF)))
                h01 = [(b[0] if (lo & 1) else a[0]) ^
                       (b[1] if (lo >> 1) else a[1]) for lo in range(4)]
                w01 = [(fr[0] if (lo & 1) else om[0]) *
                       (fr[1] if (lo >> 1) else om[1]) for lo in range(4)]
                h23 = [(b[2] if (hi & 1) else a[2]) ^
                       (b[3] if (hi >> 1) else a[3]) for hi in range(4)]
                w23 = [(fr[2] if (hi & 1) else om[2]) *
                       (fr[3] if (hi >> 1) else om[3]) for hi in range(4)]
                acc0 = jnp.zeros((16,), jnp.float32)
                acc1 = jnp.zeros((16,), jnp.float32)
                for cj in range(16):
                    idx = h01[cj & 3] ^ h23[cj >> 2]
                    w = w01[cj & 3] * w23[cj >> 2]
                    word = plsc.load_gather(tab_v, [idx])
                    f0 = plsc.bitcast(lax.shift_left(word, jnp.int32(16)),
                                      jnp.float32)
                    f1 = plsc.bitcast(lax.bitwise_and(word, jnp.int32(-65536)),
                                      jnp.float32)
                    acc0 = acc0 + w * f0
                    acc1 = acc1 + w * f1
                o0_v[pl.ds(off, 16)] = acc0
                o1_v[pl.ds(off, 16)] = acc1

            pltpu.sync_copy(o0_v, out_hbm.at[task * 2, pl.ds(c * CS, CS)])
            pltpu.sync_copy(o1_v, out_hbm.at[task * 2 + 1, pl.ds(c * CS, CS)])


_sc_encode = functools.partial(
    pl.kernel,
    out_type=jax.ShapeDtypeStruct((128, B), jnp.float32),
    mesh=plsc.VectorSubcoreMesh(core_axis_name="c", subcore_axis_name="s"),
    compiler_params=pltpu.CompilerParams(needs_layout_passes=False),
    scratch_types=[
        pltpu.VMEM((T,), jnp.int32),
        pltpu.VMEM((4, CS), jnp.float32),
        pltpu.VMEM((CS,), jnp.float32),
        pltpu.VMEM((CS,), jnp.float32),
    ],
)(_sc_encode_body)


def _mlp_body(e_ref, w1_ref, b1_ref, w2_ref, b2_ref, o_ref):
    e = e_ref[...]
    h = lax.dot_general(w1_ref[...], e, (((0,), (0,)), ((), ())),
                        preferred_element_type=jnp.float32)
    h = h + b1_ref[...]
    h = jnp.where(h >= 0, h, 0.01 * h)
    lt = lax.dot_general(w2_ref[...], h, (((0,), (0,)), ((), ())),
                         preferred_element_type=jnp.float32)
    lt = lt + b2_ref[...]
    o_ref[...] = lt.T


def _mlp(enc, W1, b1c, W2, b2c):
    bsz = 2048
    return pl.pallas_call(
        _mlp_body,
        grid=(B // bsz,),
        in_specs=[
            pl.BlockSpec((128, bsz), lambda i: (0, i)),
            pl.BlockSpec((128, 256), lambda i: (0, 0)),
            pl.BlockSpec((256, 1), lambda i: (0, 0)),
            pl.BlockSpec((256, 64), lambda i: (0, 0)),
            pl.BlockSpec((64, 1), lambda i: (0, 0)),
        ],
        out_specs=pl.BlockSpec((bsz, 64), lambda i: (i, 0)),
        out_shape=jax.ShapeDtypeStruct((B, 64), jnp.float32),
    )(enc, W1, b1c, W2, b2c)


def kernel(z, tables, W1, b1, W2, b2):
    zt = z.T
    tabp = lax.bitcast_convert_type(
        tables.astype(jnp.bfloat16).reshape(64, T, 2), jnp.int32)
    enc = _sc_encode(tabp, zt)
    return _mlp(enc, W1, b1.reshape(256, 1), W2, b2.reshape(64, 1))


# speedup vs baseline: 2.2466x; 1.0401x over previous
import functools

import numpy as np
import jax
import jax.numpy as jnp
from jax import lax
from jax.experimental import pallas as pl
from jax.experimental.pallas import tpu as pltpu
from jax.experimental.pallas import tpu_sc as plsc

L = 16
T = 65536
B = 16384
PRIMES_I32 = [int(np.uint32(p).astype(np.int32)) for p in
              (1, 2654435761, 805459861, 3674653429)]
RES_LIST = [float(np.floor(16.0 * 1.5 ** l)) for l in range(L)]

NC, NS = 2, 16
NW = NC * NS
TASKS_PER_TILE = 64 // NW
CS = 4096
NSTEP = CS // 16
NCH = B // CS


def _sc_encode_body(tabp_hbm, zt_hbm, out_hbm,
                    tab_v, z2_v, o2_v,
                    sem_t, sem_z0, sem_z1, sem_o0, sem_o1):
    wid = lax.axis_index("s") * NC + lax.axis_index("c")
    sem_z = [sem_z0, sem_z1]
    sem_o = [sem_o0, sem_o1]
    o_pending = [None, None]

    for j in range(TASKS_PER_TILE):
        task = wid * TASKS_PER_TILE + j
        grp = lax.shift_right_logical(task, 4)
        lvl = lax.bitwise_and(task, 15)
        res = jnp.float32(0.0)
        for k in range(L):
            res = jnp.where(lvl == k, jnp.float32(RES_LIST[k]), res)

        tcopy = pltpu.async_copy(tabp_hbm.at[task], tab_v, sem_t)
        zcopy = [None, None]
        zcopy[0] = pltpu.async_copy(
            zt_hbm.at[pl.ds(grp * 4, 4), pl.ds(0, CS)], z2_v.at[0],
            sem_z[0])
        tcopy.wait()

        for c in range(NCH):
            cur = c & 1
            if c + 1 < NCH:
                zcopy[1 - cur] = pltpu.async_copy(
                    zt_hbm.at[pl.ds(grp * 4, 4), pl.ds((c + 1) * CS, CS)],
                    z2_v.at[1 - cur], sem_z[1 - cur])
            zcopy[cur].wait()
            if o_pending[cur] is not None:
                o_pending[cur].wait()
                o_pending[cur] = None

            @plsc.parallel_loop(0, NSTEP, unroll=2)
            def step(s):
                off = pl.multiple_of(s * 16, 16)
                fr = []
                om = []
                a = []
                b = []
                for dd in range(4):
                    zd = z2_v[cur, dd, pl.ds(off, 16)]
                    x = 1.0 / (1.0 + jnp.exp(-zd))
                    pos = x * res
                    pi = pos.astype(jnp.int32)
                    fd = pos - pi.astype(jnp.float32)
                    fr.append(fd)
                    om.append(1.0 - fd)
                    if dd == 0:
                        ad = pi
                        bd = pi + 1
                    else:
                        ad = pi * jnp.int32(PRIMES_I32[dd])
                        bd = ad + jnp.int32(PRIMES_I32[dd])
                    a.append(lax.bitwise_and(ad, jnp.int32(0xFFFF)))
                    b.append(lax.bitwise_and(bd, jnp.int32(0xFFFF)))
                h01 = [(b[0] if (lo & 1) else a[0]) ^
                       (b[1] if (lo >> 1) else a[1]) for lo in range(4)]
                w01 = [(fr[0] if (lo & 1) else om[0]) *
                       (fr[1] if (lo >> 1) else om[1]) for lo in range(4)]
                h23 = [(b[2] if (hi & 1) else a[2]) ^
                       (b[3] if (hi >> 1) else a[3]) for hi in range(4)]
                w23 = [(fr[2] if (hi & 1) else om[2]) *
                       (fr[3] if (hi >> 1) else om[3]) for hi in range(4)]
                acc0 = jnp.zeros((16,), jnp.float32)
                acc1 = jnp.zeros((16,), jnp.float32)
                for cj in range(16):
                    idx = h01[cj & 3] ^ h23[cj >> 2]
                    w = w01[cj & 3] * w23[cj >> 2]
                    word = plsc.load_gather(tab_v, [idx])
                    f0 = plsc.bitcast(lax.shift_left(word, jnp.int32(16)),
                                      jnp.float32)
                    f1 = plsc.bitcast(lax.bitwise_and(word, jnp.int32(-65536)),
                                      jnp.float32)
                    acc0 = acc0 + w * f0
                    acc1 = acc1 + w * f1
                o2_v[cur, 0, pl.ds(off, 16)] = acc0
                o2_v[cur, 1, pl.ds(off, 16)] = acc1

            o_pending[cur] = pltpu.async_copy(
                o2_v.at[cur],
                out_hbm.at[pl.ds(task * 2, 2), pl.ds(c * CS, CS)],
                sem_o[cur])

    for cur in range(2):
        if o_pending[cur] is not None:
            o_pending[cur].wait()


_sc_encode = functools.partial(
    pl.kernel,
    out_type=jax.ShapeDtypeStruct((128, B), jnp.float32),
    mesh=plsc.VectorSubcoreMesh(core_axis_name="c", subcore_axis_name="s"),
    compiler_params=pltpu.CompilerParams(needs_layout_passes=False),
    scratch_types=[
        pltpu.VMEM((T,), jnp.int32),
        pltpu.VMEM((2, 4, CS), jnp.float32),
        pltpu.VMEM((2, 2, CS), jnp.float32),
        pltpu.SemaphoreType.DMA,
        pltpu.SemaphoreType.DMA,
        pltpu.SemaphoreType.DMA,
        pltpu.SemaphoreType.DMA,
        pltpu.SemaphoreType.DMA,
    ],
)(_sc_encode_body)


def _mlp_body(e_ref, w1_ref, b1_ref, w2_ref, b2_ref, o_ref):
    e = e_ref[...]
    h = lax.dot_general(w1_ref[...], e, (((0,), (0,)), ((), ())),
                        preferred_element_type=jnp.float32)
    h = h + b1_ref[...]
    h = jnp.where(h >= 0, h, 0.01 * h)
    lt = lax.dot_general(w2_ref[...], h, (((0,), (0,)), ((), ())),
                         preferred_element_type=jnp.float32)
    lt = lt + b2_ref[...]
    o_ref[...] = lt.T


def _mlp(enc, W1, b1c, W2, b2c):
    bsz = 2048
    return pl.pallas_call(
        _mlp_body,
        grid=(B // bsz,),
        in_specs=[
            pl.BlockSpec((128, bsz), lambda i: (0, i)),
            pl.BlockSpec((128, 256), lambda i: (0, 0)),
            pl.BlockSpec((256, 1), lambda i: (0, 0)),
            pl.BlockSpec((256, 64), lambda i: (0, 0)),
            pl.BlockSpec((64, 1), lambda i: (0, 0)),
        ],
        out_specs=pl.BlockSpec((bsz, 64), lambda i: (i, 0)),
        out_shape=jax.ShapeDtypeStruct((B, 64), jnp.float32),
    )(enc, W1, b1c, W2, b2c)


def kernel(z, tables, W1, b1, W2, b2):
    zt = z.T
    tabp = lax.bitcast_convert_type(
        tables.astype(jnp.bfloat16).reshape(64, T, 2), jnp.int32)
    enc = _sc_encode(tabp, zt)
    return _mlp(enc, W1, b1.reshape(256, 1), W2, b2.reshape(64, 1))


# speedup vs baseline: 2.2488x; 1.0010x over previous
import functools

import numpy as np
import jax
import jax.numpy as jnp
from jax import lax
from jax.experimental import pallas as pl
from jax.experimental.pallas import tpu as pltpu
from jax.experimental.pallas import tpu_sc as plsc

L = 16
T = 65536
B = 16384
PRIMES_I32 = [int(np.uint32(p).astype(np.int32)) for p in
              (1, 2654435761, 805459861, 3674653429)]
RES_LIST = [float(np.floor(16.0 * 1.5 ** l)) for l in range(L)]

NC, NS = 2, 16
NW = NC * NS
TASKS_PER_TILE = 64 // NW
CS = 4096
NSTEP = CS // 16
NCH = B // CS


def _sc_encode_body(tabb_hbm, zt_hbm, out_hbm,
                    tab_v, z2_v, o2_v,
                    sem_t, sem_z0, sem_z1, sem_o0, sem_o1):
    wid = lax.axis_index("s") * NC + lax.axis_index("c")
    tabp_hbm = tabb_hbm
    sem_z = [sem_z0, sem_z1]
    sem_o = [sem_o0, sem_o1]
    o_pending = [None, None]

    for j in range(TASKS_PER_TILE):
        task = wid * TASKS_PER_TILE + j
        grp = lax.shift_right_logical(task, 4)
        lvl = lax.bitwise_and(task, 15)
        res = jnp.float32(0.0)
        for k in range(L):
            res = jnp.where(lvl == k, jnp.float32(RES_LIST[k]), res)

        tcopy = pltpu.async_copy(tabp_hbm.at[task], tab_v, sem_t)
        zcopy = [None, None]
        zcopy[0] = pltpu.async_copy(
            zt_hbm.at[pl.ds(grp * 4, 4), pl.ds(0, CS)], z2_v.at[0],
            sem_z[0])
        tcopy.wait()

        for c in range(NCH):
            cur = c & 1
            if c + 1 < NCH:
                zcopy[1 - cur] = pltpu.async_copy(
                    zt_hbm.at[pl.ds(grp * 4, 4), pl.ds((c + 1) * CS, CS)],
                    z2_v.at[1 - cur], sem_z[1 - cur])
            zcopy[cur].wait()
            if o_pending[cur] is not None:
                o_pending[cur].wait()
                o_pending[cur] = None

            @plsc.parallel_loop(0, NSTEP, unroll=2)
            def step(s):
                off = pl.multiple_of(s * 16, 16)
                fr = []
                om = []
                a = []
                b = []
                for dd in range(4):
                    zd = z2_v[cur, dd, pl.ds(off, 16)]
                    x = 1.0 / (1.0 + jnp.exp(-zd))
                    pos = x * res
                    pi = pos.astype(jnp.int32)
                    fd = pos - pi.astype(jnp.float32)
                    fr.append(fd)
                    om.append(1.0 - fd)
                    if dd == 0:
                        ad = pi
                        bd = pi + 1
                    else:
                        ad = pi * jnp.int32(PRIMES_I32[dd])
                        bd = ad + jnp.int32(PRIMES_I32[dd])
                    a.append(lax.bitwise_and(ad, jnp.int32(0xFFFF)))
                    b.append(lax.bitwise_and(bd, jnp.int32(0xFFFF)))
                h01 = [(b[0] if (lo & 1) else a[0]) ^
                       (b[1] if (lo >> 1) else a[1]) for lo in range(4)]
                w01 = [(fr[0] if (lo & 1) else om[0]) *
                       (fr[1] if (lo >> 1) else om[1]) for lo in range(4)]
                h23 = [(b[2] if (hi & 1) else a[2]) ^
                       (b[3] if (hi >> 1) else a[3]) for hi in range(4)]
                w23 = [(fr[2] if (hi & 1) else om[2]) *
                       (fr[3] if (hi >> 1) else om[3]) for hi in range(4)]
                acc0 = jnp.zeros((16,), jnp.float32)
                acc1 = jnp.zeros((16,), jnp.float32)
                for cj in range(16):
                    idx = h01[cj & 3] ^ h23[cj >> 2]
                    w = w01[cj & 3] * w23[cj >> 2]
                    word = plsc.load_gather(tab_v, [idx])
                    f0 = plsc.bitcast(lax.shift_left(word, jnp.int32(16)),
                                      jnp.float32)
                    f1 = plsc.bitcast(lax.bitwise_and(word, jnp.int32(-65536)),
                                      jnp.float32)
                    acc0 = acc0 + w * f0
                    acc1 = acc1 + w * f1
                o2_v[cur, 0, pl.ds(off, 16)] = acc0
                o2_v[cur, 1, pl.ds(off, 16)] = acc1

            o_pending[cur] = pltpu.async_copy(
                o2_v.at[cur],
                out_hbm.at[pl.ds(task * 2, 2), pl.ds(c * CS, CS)],
                sem_o[cur])

    for cur in range(2):
        if o_pending[cur] is not None:
            o_pending[cur].wait()


_sc_encode = functools.partial(
    pl.kernel,
    out_type=jax.ShapeDtypeStruct((128, B), jnp.float32),
    mesh=plsc.VectorSubcoreMesh(core_axis_name="c", subcore_axis_name="s"),
    compiler_params=pltpu.CompilerParams(needs_layout_passes=False),
    scratch_types=[
        pltpu.VMEM((T,), jnp.int32),
        pltpu.VMEM((2, 4, CS), jnp.float32),
        pltpu.VMEM((2, 2, CS), jnp.float32),
        pltpu.SemaphoreType.DMA,
        pltpu.SemaphoreType.DMA,
        pltpu.SemaphoreType.DMA,
        pltpu.SemaphoreType.DMA,
        pltpu.SemaphoreType.DMA,
    ],
)(_sc_encode_body)


def _mlp_body(e_ref, w1_ref, b1_ref, w2_ref, b2_ref, o_ref):
    e = e_ref[...]
    h = lax.dot_general(w1_ref[...], e, (((0,), (0,)), ((), ())),
                        preferred_element_type=jnp.float32)
    h = h + b1_ref[...]
    h = jnp.where(h >= 0, h, 0.01 * h)
    lt = lax.dot_general(w2_ref[...], h, (((0,), (0,)), ((), ())),
                         preferred_element_type=jnp.float32)
    lt = lt + b2_ref[...]
    o_ref[...] = lt.T


def _mlp(enc, W1, b1c, W2, b2c):
    bsz = 2048
    return pl.pallas_call(
        _mlp_body,
        grid=(B // bsz,),
        in_specs=[
            pl.BlockSpec((128, bsz), lambda i: (0, i)),
            pl.BlockSpec((128, 256), lambda i: (0, 0)),
            pl.BlockSpec((256, 1), lambda i: (0, 0)),
            pl.BlockSpec((256, 64), lambda i: (0, 0)),
            pl.BlockSpec((64, 1), lambda i: (0, 0)),
        ],
        out_specs=pl.BlockSpec((bsz, 64), lambda i: (i, 0)),
        out_shape=jax.ShapeDtypeStruct((B, 64), jnp.float32),
    )(enc, W1, b1c, W2, b2c)


def kernel(z, tables, W1, b1, W2, b2):
    zt = z.T
    tabb = lax.bitcast_convert_type(
        tables.reshape(64, 2 * T).astype(jnp.bfloat16).reshape(64, T, 2),
        jnp.int32)
    enc = _sc_encode(tabb, zt)
    return _mlp(enc, W1, b1.reshape(256, 1), W2, b2.reshape(64, 1))
